# baseline (speedup 1.0000x reference)
import functools

import jax
import jax.numpy as jnp
from jax.experimental import pallas as pl
from jax.experimental.pallas import tpu as pltpu

_RATIO = 0.6


def _mask_body(idx_ref, x_ref, o_ref, u_ref, *, num_mask, nb, sub, lane):
    g = pl.program_id(0)

    gidx = (jax.lax.broadcasted_iota(jnp.int32, (sub, lane), 0) * lane
            + jax.lax.broadcasted_iota(jnp.int32, (sub, lane), 1))

    x = x_ref[...]

    lane_iota = jax.lax.broadcasted_iota(jnp.int32, (1, lane), 1)
    cs = []
    for r in range(nb):
        idx = idx_ref[g * nb + r]
        si, li = idx // lane, idx % lane
        for d in range(3):
            row = x_ref[r, d, pl.ds(si, 1), :]
            cs.append(jnp.sum(jnp.where(lane_iota == li, row, 0.0)))
    c = jnp.stack(cs).reshape(nb, 3, 1, 1)

    d = x - c
    dist = jnp.sum(d * d, axis=1)
    u_ref[...] = jax.lax.bitcast_convert_type(dist, jnp.int32)

    k = jnp.int32(num_mask)

    def vbody(i, t):
        tc = t | (jnp.int32(1) << (30 - i))
        cnt = jnp.sum((u_ref[...] < tc).astype(jnp.int32), axis=(1, 2),
                      keepdims=True)
        return jnp.where(cnt < k, tc, t)

    t = jax.lax.fori_loop(0, 31, vbody, jnp.zeros((nb, 1, 1), jnp.int32))

    at_or_below = jnp.sum((u_ref[...] <= t).astype(jnp.int32), axis=(1, 2),
                          keepdims=True)

    def no_ties():
        return jnp.full((nb, 1, 1), jnp.int32(0x7FFFFFFF))

    def break_ties():
        below = jnp.sum((u_ref[...] < t).astype(jnp.int32), axis=(1, 2),
                        keepdims=True)
        need = k - below

        def ibody(i, a):
            ac = a | (jnp.int32(1) << (15 - i))
            f = jnp.sum(
                ((u_ref[...] == t) & (gidx[None] < ac)).astype(jnp.int32),
                axis=(1, 2), keepdims=True)
            return jnp.where(f <= need, ac, a)

        return jax.lax.fori_loop(0, 16, ibody,
                                 jnp.zeros((nb, 1, 1), jnp.int32))

    ia = jax.lax.cond(jnp.all(at_or_below <= k), no_ties, break_ties)

    o_ref[...] = (u_ref[...] < t) | ((u_ref[...] == t) & (gidx[None] < ia))


def _half(centers_h, rand_idx_h, num_mask, sub, lane, nb):
    bh = centers_h.shape[0]
    nb = min(nb, bh)
    ct = jnp.transpose(centers_h, (0, 2, 1)).reshape(bh, 3, sub, lane)
    body = functools.partial(_mask_body, num_mask=num_mask, nb=nb, sub=sub,
                             lane=lane)
    return pl.pallas_call(
        body,
        grid_spec=pltpu.PrefetchScalarGridSpec(
            num_scalar_prefetch=1,
            grid=(bh // nb,),
            in_specs=[
                pl.BlockSpec((nb, 3, sub, lane), lambda i, idx: (i, 0, 0, 0)),
            ],
            out_specs=pl.BlockSpec((nb, sub, lane), lambda i, idx: (i, 0, 0)),
            scratch_shapes=[pltpu.VMEM((nb, sub, lane), jnp.int32)],
        ),
        out_shape=jax.ShapeDtypeStruct((bh, sub, lane), jnp.bool_),
    )(rand_idx_h, ct)


def kernel(centers):
    b, g, dim = centers.shape
    assert dim == 3
    num_mask = int(_RATIO * g)
    if num_mask == 0:
        return jnp.zeros((b, g), dtype=jnp.bool_)

    key = jax.random.key(42)
    rand_idx = jax.random.randint(key, (b, 1), 0, g)[:, 0].astype(jnp.int32)

    sub = 8
    lane = g // sub
    nb = 16
    nhalves = 4
    bh = b // nhalves
    outs = [
        _half(centers[h * bh:(h + 1) * bh], rand_idx[h * bh:(h + 1) * bh],
              num_mask, sub, lane, nb)
        for h in range(nhalves)
    ]
    return jnp.concatenate(outs, axis=0).reshape(b, g)


# speedup vs baseline: 1.2639x; 1.2639x over previous
import functools

import jax
import jax.numpy as jnp
from jax.experimental import pallas as pl
from jax.experimental.pallas import tpu as pltpu

_RATIO = 0.6


def _mask_body(idx_ref, x_ref, o_ref, u_ref, *, num_mask, nb, sub, lane):
    g = pl.program_id(0)

    gidx = (jax.lax.broadcasted_iota(jnp.int32, (sub, lane), 0) * lane
            + jax.lax.broadcasted_iota(jnp.int32, (sub, lane), 1))

    x = x_ref[...]

    lane_iota = jax.lax.broadcasted_iota(jnp.int32, (1, lane), 1)
    cs = []
    for r in range(nb):
        idx = idx_ref[g * nb + r]
        si, li = idx // lane, idx % lane
        for d in range(3):
            row = x_ref[r, d, pl.ds(si, 1), :]
            cs.append(jnp.sum(jnp.where(lane_iota == li, row, 0.0)))
    c = jnp.stack(cs).reshape(nb, 3, 1, 1)

    d = x - c
    dist = jnp.sum(d * d, axis=1)
    u_ref[...] = jax.lax.bitcast_convert_type(dist, jnp.int32)

    k = jnp.int32(num_mask)

    def vbody(i, t):
        tc = t | (jnp.int32(1) << (30 - i))
        cnt = jnp.sum((u_ref[...] < tc).astype(jnp.int32), axis=(1, 2),
                      keepdims=True)
        return jnp.where(cnt < k, tc, t)

    t = jax.lax.fori_loop(0, 31, vbody, jnp.zeros((nb, 1, 1), jnp.int32))

    at_or_below = jnp.sum((u_ref[...] <= t).astype(jnp.int32), axis=(1, 2),
                          keepdims=True)

    def no_ties():
        return jnp.full((nb, 1, 1), jnp.int32(0x7FFFFFFF))

    def break_ties():
        below = jnp.sum((u_ref[...] < t).astype(jnp.int32), axis=(1, 2),
                        keepdims=True)
        need = k - below

        def ibody(i, a):
            ac = a | (jnp.int32(1) << (15 - i))
            f = jnp.sum(
                ((u_ref[...] == t) & (gidx[None] < ac)).astype(jnp.int32),
                axis=(1, 2), keepdims=True)
            return jnp.where(f <= need, ac, a)

        return jax.lax.fori_loop(0, 16, ibody,
                                 jnp.zeros((nb, 1, 1), jnp.int32))

    ia = jax.lax.cond(jnp.all(at_or_below <= k), no_ties, break_ties)

    o_ref[...] = (u_ref[...] < t) | ((u_ref[...] == t) & (gidx[None] < ia))


def _half(centers_h, rand_idx_h, num_mask, sub, lane, nb):
    bh = centers_h.shape[0]
    nb = min(nb, bh)
    ct = jnp.transpose(centers_h, (0, 2, 1)).reshape(bh, 3, sub, lane)
    body = functools.partial(_mask_body, num_mask=num_mask, nb=nb, sub=sub,
                             lane=lane)
    return pl.pallas_call(
        body,
        grid_spec=pltpu.PrefetchScalarGridSpec(
            num_scalar_prefetch=1,
            grid=(bh // nb,),
            in_specs=[
                pl.BlockSpec((nb, 3, sub, lane), lambda i, idx: (i, 0, 0, 0)),
            ],
            out_specs=pl.BlockSpec((nb, sub, lane), lambda i, idx: (i, 0, 0)),
            scratch_shapes=[pltpu.VMEM((nb, sub, lane), jnp.int32)],
        ),
        out_shape=jax.ShapeDtypeStruct((bh, sub, lane), jnp.bool_),
    )(rand_idx_h, ct)


def kernel(centers):
    b, g, dim = centers.shape
    assert dim == 3
    num_mask = int(_RATIO * g)
    if num_mask == 0:
        return jnp.zeros((b, g), dtype=jnp.bool_)

    key = jax.random.key(42)
    rand_idx = jax.random.randint(key, (b, 1), 0, g)[:, 0].astype(jnp.int32)

    sub = 8
    lane = g // sub
    nb = 16
    nhalves = 1
    bh = b // nhalves
    outs = [
        _half(centers[h * bh:(h + 1) * bh], rand_idx[h * bh:(h + 1) * bh],
              num_mask, sub, lane, nb)
        for h in range(nhalves)
    ]
    return jnp.concatenate(outs, axis=0).reshape(b, g)


# speedup vs baseline: 1.3316x; 1.0536x over previous
import functools

import jax
import jax.numpy as jnp
from jax.experimental import pallas as pl
from jax.experimental.pallas import tpu as pltpu

_RATIO = 0.6


def _mask_body(idx_ref, x_ref, o_ref, u_ref, *, num_mask, nb, sub, lane):
    g = pl.program_id(0)

    gidx = (jax.lax.broadcasted_iota(jnp.int32, (sub, lane), 0) * lane
            + jax.lax.broadcasted_iota(jnp.int32, (sub, lane), 1))

    x = x_ref[...]

    sel = jnp.stack([gidx == idx_ref[g * nb + r] for r in range(nb)])
    c = jnp.sum(jnp.where(sel[:, None], x, 0.0), axis=(2, 3), keepdims=True)

    d = x - c
    dist = jnp.sum(d * d, axis=1)
    u_ref[...] = jax.lax.bitcast_convert_type(dist, jnp.int32)

    k = jnp.int32(num_mask)

    def vbody(i, t):
        tc = t | (jnp.int32(1) << (30 - i))
        cnt = jnp.sum((u_ref[...] < tc).astype(jnp.int32), axis=(1, 2),
                      keepdims=True)
        return jnp.where(cnt < k, tc, t)

    t = jax.lax.fori_loop(0, 31, vbody, jnp.zeros((nb, 1, 1), jnp.int32))

    at_or_below = jnp.sum((u_ref[...] <= t).astype(jnp.int32), axis=(1, 2),
                          keepdims=True)

    def no_ties():
        return jnp.full((nb, 1, 1), jnp.int32(0x7FFFFFFF))

    def break_ties():
        below = jnp.sum((u_ref[...] < t).astype(jnp.int32), axis=(1, 2),
                        keepdims=True)
        need = k - below

        def ibody(i, a):
            ac = a | (jnp.int32(1) << (15 - i))
            f = jnp.sum(
                ((u_ref[...] == t) & (gidx[None] < ac)).astype(jnp.int32),
                axis=(1, 2), keepdims=True)
            return jnp.where(f <= need, ac, a)

        return jax.lax.fori_loop(0, 16, ibody, jnp.zeros((nb, 1, 1),
                                                         jnp.int32))

    ia = jax.lax.cond(jnp.all(at_or_below <= k), no_ties, break_ties)

    o_ref[...] = (u_ref[...] < t) | ((u_ref[...] == t) & (gidx[None] < ia))


def kernel(centers):
    b, g, d = centers.shape
    assert d == 3
    num_mask = int(_RATIO * g)
    if num_mask == 0:
        return jnp.zeros((b, g), dtype=jnp.bool_)

    key = jax.random.key(42)
    rand_idx = jax.random.randint(key, (b, 1), 0, g)[:, 0].astype(jnp.int32)

    sub = 8
    lane = g // sub
    ct = jnp.transpose(centers, (0, 2, 1)).reshape(b, 3, sub, lane)

    nb = 32
    body = functools.partial(_mask_body, num_mask=num_mask, nb=nb, sub=sub,
                             lane=lane)
    out = pl.pallas_call(
        body,
        grid_spec=pltpu.PrefetchScalarGridSpec(
            num_scalar_prefetch=1,
            grid=(b // nb,),
            in_specs=[
                pl.BlockSpec((nb, 3, sub, lane), lambda i, idx: (i, 0, 0, 0)),
            ],
            out_specs=pl.BlockSpec((nb, sub, lane), lambda i, idx: (i, 0, 0)),
            scratch_shapes=[pltpu.VMEM((nb, sub, lane), jnp.int32)],
        ),
        out_shape=jax.ShapeDtypeStruct((b, sub, lane), jnp.bool_),
    )(rand_idx, ct)
    return out.reshape(b, g)


# speedup vs baseline: 1.3319x; 1.0002x over previous
import functools

import jax
import jax.numpy as jnp
from jax.experimental import pallas as pl
from jax.experimental.pallas import tpu as pltpu

_RATIO = 0.6


def _mask_body(idx_ref, x_ref, o_ref, u_ref, *, num_mask, nb, sub, lane):
    g = pl.program_id(0)

    gidx = (jax.lax.broadcasted_iota(jnp.int32, (sub, lane), 0) * lane
            + jax.lax.broadcasted_iota(jnp.int32, (sub, lane), 1))

    for r in range(nb):
        xr = x_ref[r]
        sel = gidx == idx_ref[g * nb + r]
        c = jnp.sum(jnp.where(sel[None], xr, 0.0), axis=(1, 2),
                    keepdims=True)
        d = xr - c
        dist = jnp.sum(d * d, axis=0)
        u_ref[r] = jax.lax.bitcast_convert_type(dist, jnp.int32)

    k = jnp.int32(num_mask)

    def vbody(i, t):
        tc = t | (jnp.int32(1) << (30 - i))
        cnt = jnp.sum((u_ref[...] < tc).astype(jnp.int32), axis=(1, 2),
                      keepdims=True)
        return jnp.where(cnt < k, tc, t)

    t = jax.lax.fori_loop(0, 31, vbody, jnp.zeros((nb, 1, 1), jnp.int32))

    at_or_below = jnp.sum((u_ref[...] <= t).astype(jnp.int32), axis=(1, 2),
                          keepdims=True)

    def no_ties():
        return jnp.full((nb, 1, 1), jnp.int32(0x7FFFFFFF))

    def break_ties():
        below = jnp.sum((u_ref[...] < t).astype(jnp.int32), axis=(1, 2),
                        keepdims=True)
        need = k - below

        def ibody(i, a):
            ac = a | (jnp.int32(1) << (15 - i))
            f = jnp.sum(
                ((u_ref[...] == t) & (gidx[None] < ac)).astype(jnp.int32),
                axis=(1, 2), keepdims=True)
            return jnp.where(f <= need, ac, a)

        return jax.lax.fori_loop(0, 16, ibody, jnp.zeros((nb, 1, 1),
                                                         jnp.int32))

    ia = jax.lax.cond(jnp.all(at_or_below <= k), no_ties, break_ties)

    o_ref[...] = (u_ref[...] < t) | ((u_ref[...] == t) & (gidx[None] < ia))


def kernel(centers):
    b, g, d = centers.shape
    assert d == 3
    num_mask = int(_RATIO * g)
    if num_mask == 0:
        return jnp.zeros((b, g), dtype=jnp.bool_)

    key = jax.random.key(42)
    rand_idx = jax.random.randint(key, (b, 1), 0, g)[:, 0].astype(jnp.int32)

    sub = 8
    lane = g // sub
    ct = jnp.transpose(centers, (0, 2, 1)).reshape(b, 3, sub, lane)

    nb = min(64, b)
    body = functools.partial(_mask_body, num_mask=num_mask, nb=nb, sub=sub,
                             lane=lane)
    out = pl.pallas_call(
        body,
        grid_spec=pltpu.PrefetchScalarGridSpec(
            num_scalar_prefetch=1,
            grid=(b // nb,),
            in_specs=[
                pl.BlockSpec((nb, 3, sub, lane), lambda i, idx: (i, 0, 0, 0)),
            ],
            out_specs=pl.BlockSpec((nb, sub, lane), lambda i, idx: (i, 0, 0)),
            scratch_shapes=[pltpu.VMEM((nb, sub, lane), jnp.int32)],
        ),
        out_shape=jax.ShapeDtypeStruct((b, sub, lane), jnp.bool_),
    )(rand_idx, ct)
    return out.reshape(b, g)
